# Initial kernel scaffold; baseline (speedup 1.0000x reference)
import jax
import jax.numpy as jnp
from jax.experimental import pallas as pl


def kernel(x_node, edge_index_ast, edge_index_df, edge_index_cf, batch_node, proj_W, proj_b, gcn_W, gcn_b, sgA_Wl, sgA_bl, sgA_Wr, sgB_Wl, sgB_bl, sgB_Wr, ln_g, ln_b, att_W, att_b, out_W, out_b):
    raise NotImplementedError("write your pallas kernel here")


# speedup vs baseline: 1.5263x; 1.5263x over previous
import jax
import jax.numpy as jnp
from jax.experimental import pallas as pl
from jax.experimental.pallas import tpu as pltpu

N = 10000
D = 256
G = 64


def _final_mm_kernel(g_ref, w_ref, b_ref, o_ref):
    o_ref[...] = g_ref[...] @ w_ref[...] + b_ref[...]


def kernel(x_node, edge_index_ast, edge_index_df, edge_index_cf, batch_node,
           proj_W, proj_b, gcn_W, gcn_b, sgA_Wl, sgA_bl, sgA_Wr,
           sgB_Wl, sgB_bl, sgB_Wr, ln_g, ln_b, att_W, att_b, out_W, out_b):
    h = jax.nn.relu(x_node @ proj_W + proj_b)
    n = h.shape[0]
    sl = jnp.arange(n, dtype=jnp.int32)

    ones_e = jnp.ones((edge_index_ast.shape[1],), jnp.float32)
    deg = jax.ops.segment_sum(ones_e, edge_index_ast[1], num_segments=n) + 1.0
    dinv = deg ** -0.5
    cntA = jnp.maximum(jax.ops.segment_sum(ones_e, edge_index_df[1], num_segments=n), 1.0)
    cntB = jnp.maximum(jax.ops.segment_sum(ones_e, edge_index_cf[1], num_segments=n), 1.0)

    for l in range(gcn_W.shape[0]):
        hp = h * dinv[:, None]
        m_ast = jax.ops.segment_sum(hp[edge_index_ast[0]], edge_index_ast[1], num_segments=n)
        a = dinv[:, None] * (m_ast + hp)
        m_df = jax.ops.segment_sum(h[edge_index_df[0]], edge_index_df[1], num_segments=n) / cntA[:, None]
        m_cf = jax.ops.segment_sum(h[edge_index_cf[0]], edge_index_cf[1], num_segments=n) / cntB[:, None]
        out = (a @ gcn_W[l] + gcn_b[l]
               + m_df @ sgA_Wl[l] + sgA_bl[l] + h @ sgA_Wr[l]
               + m_cf @ sgB_Wl[l] + sgB_bl[l] + h @ sgB_Wr[l])
        out = jax.nn.relu(out)
        mu = out.mean(-1, keepdims=True)
        var = out.var(-1, keepdims=True)
        h = (out - mu) / jnp.sqrt(var + 1e-5) * ln_g + ln_b

    scores = (h @ att_W + att_b)[:, 0]
    scores = jax.nn.softmax(scores - scores.max())
    weighted = h * scores[:, None]
    graph_repr = jax.ops.segment_sum(weighted, batch_node, num_segments=G)
    graph_counts = jax.ops.segment_sum(scores, batch_node, num_segments=G)[:, None]
    graph_repr = graph_repr / jnp.clip(graph_counts, 1e-6)

    return pl.pallas_call(
        _final_mm_kernel,
        out_shape=jax.ShapeDtypeStruct((G, D), jnp.float32),
    )(graph_repr, out_W, out_b)


# speedup vs baseline: 2.0189x; 1.3228x over previous
import functools

import jax
import jax.numpy as jnp
from jax import lax
from jax.experimental import pallas as pl
from jax.experimental.pallas import tpu as pltpu
from jax.experimental.pallas import tpu_sc as plsc

N = 10000
D = 256
G = 64
E = 160000
NB = 32
R = 320
NR = NB * R
RT = R + 8
K = 128
C = 2000
EPAD = E + K

_mesh = plsc.VectorSubcoreMesh(core_axis_name="c", subcore_axis_name="s",
                               num_cores=2, num_subcores=16)

_i32 = jnp.int32
_f32 = jnp.float32


@functools.partial(
    pl.kernel,
    out_type=(
        jax.ShapeDtypeStruct((NB, EPAD), _i32),
        jax.ShapeDtypeStruct((NB, EPAD), _i32),
        jax.ShapeDtypeStruct((NB, EPAD), _i32),
        jax.ShapeDtypeStruct((NB, 16), _i32),
        jax.ShapeDtypeStruct((NB, 16), _i32),
        jax.ShapeDtypeStruct((NB, 16), _i32),
        jax.ShapeDtypeStruct((NR, 16), _f32),
        jax.ShapeDtypeStruct((NR, 16), _f32),
        jax.ShapeDtypeStruct((NR, 16), _f32),
    ),
    mesh=_mesh,
    scratch_types=[
        pltpu.VMEM((C,), _i32),
        pltpu.VMEM((C,), _i32),
        pltpu.VMEM((C + 304,), _i32),
        pltpu.VMEM((16,), _i32),
        pltpu.VMEM((K + 16,), _i32),
        pltpu.VMEM((RT, 16), _f32),
    ],
    compiler_params=pltpu.CompilerParams(needs_layout_passes=False),
)
def _sc_bucketize(es0, ed0, es1, ed1, es2, ed2, lst0, lst1, lst2,
                  cnt0, cnt1, cnt2, deg0, deg1, deg2, srcb, dstb, listb,
                  cvec, kbuf, acc16):
    c = lax.axis_index("c")
    s = lax.axis_index("s")
    w = c * 16 + s
    lo = w * R

    for es, ed, lst, cnt, deg in ((es0, ed0, lst0, cnt0, deg0),
                                  (es1, ed1, lst1, cnt1, deg1),
                                  (es2, ed2, lst2, cnt2, deg2)):
        def chunk_body(t, carry):
            pos, outb = carry
            pltpu.sync_copy(es.at[pl.ds(t * C, C)], srcb)
            pltpu.sync_copy(ed.at[pl.ds(t * C, C)], dstb)

            def vec_body(j, pos):
                sv = srcb[pl.ds(j * 16, 16)]
                dv = dstb[pl.ds(j * 16, 16)]
                dl = dv - lo
                m = (dl >= 0) & (dl < R)
                pk = (sv << 9) | jnp.where(m, dl, 0)
                kin = m.astype(_i32)
                excl = plsc.cumsum(kin) - kin
                idx = jnp.where(m, pos + excl, C + 303)
                plsc.store_scatter(listb, [idx], pk)
                return pos + jnp.sum(kin)
            pos = lax.fori_loop(0, C // 16, vec_body, pos)

            nblk = pos // K

            def flush(b, _):
                pltpu.sync_copy(listb.at[pl.ds(b * K, K)],
                                lst.at[w, pl.ds((outb + b) * K, K)])
                return 0
            lax.fori_loop(0, nblk, flush, 0)
            rs = nblk * K

            def shift(v, _):
                tmp = listb[pl.ds(rs + v * 16, 16)]
                listb[pl.ds(v * 16, 16)] = tmp
                return 0
            lax.fori_loop(0, K // 16, shift, 0)
            return pos - rs, outb + nblk

        pos, outb = lax.fori_loop(0, E // C, chunk_body,
                                  (jnp.asarray(0, _i32), jnp.asarray(0, _i32)))

        sent = jnp.full((16,), R, _i32)

        def pad(v, _):
            listb[pl.ds(pos + v * 16, 16)] = sent
            return 0
        lax.fori_loop(0, K // 16, pad, 0)
        nblk2 = (pos + K - 1) // K

        def flush2(b, _):
            pltpu.sync_copy(listb.at[pl.ds(b * K, K)],
                            lst.at[w, pl.ds((outb + b) * K, K)])
            return 0
        lax.fori_loop(0, nblk2, flush2, 0)
        total = (outb + nblk2) * K
        cvec[...] = jnp.zeros((16,), _i32) + total
        pltpu.sync_copy(cvec, cnt.at[w])

        def za16(i, _):
            acc16[i, pl.ds(0, 16)] = jnp.zeros((16,), _f32)
            return 0
        lax.fori_loop(0, RT, za16, 0)

        def count_chunk(t, _):
            pltpu.sync_copy(lst.at[w, pl.ds(t * K, K)], kbuf.at[pl.ds(0, K)])

            def one(k, _):
                dl = kbuf[pl.ds(k, 16)][0] & 511
                plsc.addupdate(acc16.at[dl, pl.ds(0, 16)],
                               jnp.ones((16,), _f32))
                return 0
            lax.fori_loop(0, K, one, 0)
            return 0
        lax.fori_loop(0, total // K, count_chunk, 0)
        pltpu.sync_copy(acc16.at[pl.ds(0, R)], deg.at[pl.ds(w * R, R)])


@functools.partial(
    pl.kernel,
    out_type=(
        jax.ShapeDtypeStruct((NR, D), _f32),
        jax.ShapeDtypeStruct((NR, D), _f32),
        jax.ShapeDtypeStruct((NR, D), _f32),
    ),
    mesh=_mesh,
    scratch_types=[
        pltpu.VMEM((K + 16,), _i32),
        pltpu.VMEM((K,), _i32),
        pltpu.VMEM((K,), _i32),
        pltpu.VMEM((K, D), _f32),
        pltpu.VMEM((16,), _i32),
        pltpu.VMEM((RT, D), _f32),
        pltpu.SemaphoreType.DMA,
    ],
    compiler_params=pltpu.CompilerParams(needs_layout_passes=False),
)
def _sc_segsum3(h, hp, lst0, lst1, lst2, cnt0, cnt1, cnt2,
                o0, o1, o2, pkb, gsrc, gdst, rows, cntv, acc, gsem):
    c = lax.axis_index("c")
    s = lax.axis_index("s")
    w = c * 16 + s

    for tab, lst, cnt, out in ((hp, lst0, cnt0, o0),
                               (h, lst1, cnt1, o1),
                               (h, lst2, cnt2, o2)):
        def za(i, _):
            acc[i // 16, pl.ds((i % 16) * 16, 16)] = jnp.zeros((16,), _f32)
            return 0
        lax.fori_loop(0, RT * 16, za, 0)

        pltpu.sync_copy(cnt.at[w], cntv)
        trips = cntv[...][0] // K

        def edge_chunk(t, _):
            pltpu.sync_copy(lst.at[w, pl.ds(t * K, K)], pkb.at[pl.ds(0, K)])

            def up(j, _):
                pk = pkb[pl.ds(j * 16, 16)]
                gsrc[pl.ds(j * 16, 16)] = pk >> 9
                return 0
            lax.fori_loop(0, K // 16, up, 0)
            pltpu.async_copy(tab.at[gsrc], rows, gsem).wait()

            def one(k, _):
                dl = pkb[pl.ds(k, 16)][0] & 511
                for cc in range(16):
                    plsc.addupdate(acc.at[dl, pl.ds(cc * 16, 16)],
                                   rows[k, pl.ds(cc * 16, 16)])
                return 0
            lax.fori_loop(0, K, one, 0)
            return 0
        lax.fori_loop(0, trips, edge_chunk, 0)

        pltpu.sync_copy(acc.at[pl.ds(0, R)], out.at[pl.ds(w * R, R)])


def _prolog_body(x_ref, w_ref, b_ref, dga_ref, ca_ref, cb_ref,
                 h_ref, hp_ref, dinv_ref, ia_ref, ib_ref):
    h = jnp.maximum(jnp.dot(x_ref[...], w_ref[...],
                            preferred_element_type=_f32) + b_ref[...], 0.0)
    dinv = lax.rsqrt(dga_ref[...] + 1.0)
    h_ref[...] = h
    hp_ref[...] = h * dinv[:, None]
    dinv_ref[...] = dinv
    ia_ref[...] = 1.0 / jnp.maximum(ca_ref[...], 1.0)
    ib_ref[...] = 1.0 / jnp.maximum(cb_ref[...], 1.0)


def _tc_prolog(x, w, b, dga, ca, cb):
    blk = 256
    grid = pl.cdiv(N, blk)
    vspec = pl.BlockSpec((blk,), lambda i: (i,))
    return pl.pallas_call(
        _prolog_body,
        grid=(grid,),
        in_specs=[
            pl.BlockSpec((blk, D), lambda i: (i, 0)),
            pl.BlockSpec((D, D), lambda i: (0, 0)),
            pl.BlockSpec((D,), lambda i: (0,)),
            vspec, vspec, vspec,
        ],
        out_specs=[
            pl.BlockSpec((blk, D), lambda i: (i, 0)),
            pl.BlockSpec((blk, D), lambda i: (i, 0)),
            vspec, vspec, vspec,
        ],
        out_shape=[
            jax.ShapeDtypeStruct((N, D), _f32),
            jax.ShapeDtypeStruct((N, D), _f32),
            jax.ShapeDtypeStruct((N,), _f32),
            jax.ShapeDtypeStruct((N,), _f32),
            jax.ShapeDtypeStruct((N,), _f32),
        ],
    )(x, w, b, dga, ca, cb)


def _layer_body(ma_ref, md_ref, mc_ref, h_ref, hp_ref, dinv_ref, ia_ref,
                ib_ref, u_ref, bias_ref, g_ref, lb_ref, hn_ref, hpn_ref):
    dinv = dinv_ref[...][:, None]
    a = (ma_ref[...] + hp_ref[...]) * dinv
    b = md_ref[...] * ia_ref[...][:, None]
    cc = mc_ref[...] * ib_ref[...][:, None]
    x = jnp.concatenate([a, b, cc, h_ref[...]], axis=1)
    out = jnp.dot(x, u_ref[...], preferred_element_type=_f32) + bias_ref[...]
    out = jnp.maximum(out, 0.0)
    mu = jnp.mean(out, axis=1, keepdims=True)
    var = jnp.mean((out - mu) ** 2, axis=1, keepdims=True)
    hn = (out - mu) * lax.rsqrt(var + 1e-5) * g_ref[...] + lb_ref[...]
    hn_ref[...] = hn
    hpn_ref[...] = hn * dinv


def _tc_layer(ma, md, mc, h, hp, dinv, ia, ib, u, bias, g, lb):
    blk = 256
    grid = pl.cdiv(N, blk)
    mspec = pl.BlockSpec((blk, D), lambda i: (i, 0))
    vspec = pl.BlockSpec((blk,), lambda i: (i,))
    return pl.pallas_call(
        _layer_body,
        grid=(grid,),
        in_specs=[
            mspec, mspec, mspec, mspec, mspec,
            vspec, vspec, vspec,
            pl.BlockSpec((4 * D, D), lambda i: (0, 0)),
            pl.BlockSpec((D,), lambda i: (0,)),
            pl.BlockSpec((D,), lambda i: (0,)),
            pl.BlockSpec((D,), lambda i: (0,)),
        ],
        out_specs=[mspec, mspec],
        out_shape=[
            jax.ShapeDtypeStruct((N, D), _f32),
            jax.ShapeDtypeStruct((N, D), _f32),
        ],
    )(ma, md, mc, h, hp, dinv, ia, ib, u, bias, g, lb)


def _epi_body(h_ref, bn_ref, attw_ref, outw_ref, outb_ref, o_ref):
    h = h_ref[...]
    logits = jnp.dot(h, attw_ref[...], preferred_element_type=_f32)[:, 0]
    p = jnp.exp(logits - jnp.max(logits))
    p = p / jnp.sum(p)
    wtd = h * p[:, None]
    oh = (bn_ref[...][:, None] ==
          lax.broadcasted_iota(_i32, (N, G), 1)).astype(_f32)
    gr = lax.dot_general(oh, wtd, (((0,), (0,)), ((), ())),
                         preferred_element_type=_f32)
    gc = lax.dot_general(oh, p[:, None], (((0,), (0,)), ((), ())),
                         preferred_element_type=_f32)
    gr = gr / jnp.maximum(gc, 1e-6)
    o_ref[...] = jnp.dot(gr, outw_ref[...],
                         preferred_element_type=_f32) + outb_ref[...]


def _tc_epilogue(h, bn, attw, outw, outb):
    return pl.pallas_call(
        _epi_body,
        out_shape=jax.ShapeDtypeStruct((G, D), _f32),
    )(h, bn, attw, outw, outb)


def kernel(x_node, edge_index_ast, edge_index_df, edge_index_cf, batch_node,
           proj_W, proj_b, gcn_W, gcn_b, sgA_Wl, sgA_bl, sgA_Wr,
           sgB_Wl, sgB_bl, sgB_Wr, ln_g, ln_b, att_W, att_b, out_W, out_b):
    (lst0, lst1, lst2, cnt0, cnt1, cnt2,
     deg0, deg1, deg2) = _sc_bucketize(
         edge_index_ast[0], edge_index_ast[1],
         edge_index_df[0], edge_index_df[1],
         edge_index_cf[0], edge_index_cf[1])
    dga = deg0[:N, 0]
    ca = deg1[:N, 0]
    cb = deg2[:N, 0]
    import os as _os
    if _os.environ.get("_BISECT") == "A":
        return (dga[:G, None] + ca[:G, None] + cb[:G, None]
                + jnp.zeros((G, D), _f32)
                + lst0[0, 0] + cnt0[0, 0])

    h, hp, dinv, ia, ib = _tc_prolog(x_node, proj_W, proj_b, dga, ca, cb)

    for l in range(gcn_W.shape[0]):
        u = jnp.concatenate(
            [gcn_W[l], sgA_Wl[l], sgB_Wl[l], sgA_Wr[l] + sgB_Wr[l]], axis=0)
        bias = gcn_b[l] + sgA_bl[l] + sgB_bl[l]
        ma, md, mc = _sc_segsum3(h, hp, lst0, lst1, lst2, cnt0, cnt1, cnt2)
        h, hp = _tc_layer(ma[:N], md[:N], mc[:N], h, hp, dinv, ia, ib,
                          u, bias, ln_g, ln_b)

    return _tc_epilogue(h, batch_node, att_W, out_W, out_b)


# speedup vs baseline: 2.0897x; 1.0350x over previous
import functools

import jax
import jax.numpy as jnp
from jax import lax
from jax.experimental import pallas as pl
from jax.experimental.pallas import tpu as pltpu
from jax.experimental.pallas import tpu_sc as plsc

N = 10000
D = 256
G = 64
E = 160000
NB = 32
R = 320
NR = NB * R
RT = R + 8
K = 128
C = 2000
EPAD = E + K

_mesh = plsc.VectorSubcoreMesh(core_axis_name="c", subcore_axis_name="s",
                               num_cores=2, num_subcores=16)

_i32 = jnp.int32
_f32 = jnp.float32


@functools.partial(
    pl.kernel,
    out_type=(
        jax.ShapeDtypeStruct((NB, EPAD), _i32),
        jax.ShapeDtypeStruct((NB, EPAD), _i32),
        jax.ShapeDtypeStruct((NB, EPAD), _i32),
        jax.ShapeDtypeStruct((NB, 16), _i32),
        jax.ShapeDtypeStruct((NB, 16), _i32),
        jax.ShapeDtypeStruct((NB, 16), _i32),
        jax.ShapeDtypeStruct((NR, 16), _f32),
        jax.ShapeDtypeStruct((NR, 16), _f32),
        jax.ShapeDtypeStruct((NR, 16), _f32),
    ),
    mesh=_mesh,
    scratch_types=[
        pltpu.VMEM((C,), _i32),
        pltpu.VMEM((C,), _i32),
        pltpu.VMEM((C + 304,), _i32),
        pltpu.VMEM((16,), _i32),
        pltpu.VMEM((K + 16,), _i32),
        pltpu.VMEM((RT, 16), _f32),
    ],
    compiler_params=pltpu.CompilerParams(needs_layout_passes=False),
)
def _sc_bucketize(es0, ed0, es1, ed1, es2, ed2, lst0, lst1, lst2,
                  cnt0, cnt1, cnt2, deg0, deg1, deg2, srcb, dstb, listb,
                  cvec, kbuf, acc16):
    c = lax.axis_index("c")
    s = lax.axis_index("s")
    w = c * 16 + s
    lo = w * R

    for es, ed, lst, cnt, deg in ((es0, ed0, lst0, cnt0, deg0),
                                  (es1, ed1, lst1, cnt1, deg1),
                                  (es2, ed2, lst2, cnt2, deg2)):
        def chunk_body(t, carry):
            pos, outb = carry
            pltpu.sync_copy(es.at[pl.ds(t * C, C)], srcb)
            pltpu.sync_copy(ed.at[pl.ds(t * C, C)], dstb)

            def vec_body(j, pos):
                sv = srcb[pl.ds(j * 16, 16)]
                dv = dstb[pl.ds(j * 16, 16)]
                dl = dv - lo
                m = (dl >= 0) & (dl < R)
                pk = (sv << 9) | jnp.where(m, dl, 0)
                kin = m.astype(_i32)
                excl = plsc.cumsum(kin) - kin
                idx = jnp.where(m, pos + excl, C + 303)
                plsc.store_scatter(listb, [idx], pk)
                return pos + jnp.sum(kin)
            pos = lax.fori_loop(0, C // 16, vec_body, pos)

            nblk = pos // K

            def flush(b, _):
                pltpu.sync_copy(listb.at[pl.ds(b * K, K)],
                                lst.at[w, pl.ds((outb + b) * K, K)])
                return 0
            lax.fori_loop(0, nblk, flush, 0)
            rs = nblk * K

            def shift(v, _):
                tmp = listb[pl.ds(rs + v * 16, 16)]
                listb[pl.ds(v * 16, 16)] = tmp
                return 0
            lax.fori_loop(0, K // 16, shift, 0)
            return pos - rs, outb + nblk

        pos, outb = lax.fori_loop(0, E // C, chunk_body,
                                  (jnp.asarray(0, _i32), jnp.asarray(0, _i32)))

        sent = jnp.full((16,), R, _i32)

        def pad(v, _):
            listb[pl.ds(pos + v * 16, 16)] = sent
            return 0
        lax.fori_loop(0, K // 16, pad, 0)
        nblk2 = (pos + K - 1) // K

        def flush2(b, _):
            pltpu.sync_copy(listb.at[pl.ds(b * K, K)],
                            lst.at[w, pl.ds((outb + b) * K, K)])
            return 0
        lax.fori_loop(0, nblk2, flush2, 0)
        total = (outb + nblk2) * K
        cvec[...] = jnp.zeros((16,), _i32) + total
        pltpu.sync_copy(cvec, cnt.at[w])

        def za16(i, _):
            acc16[i, pl.ds(0, 16)] = jnp.zeros((16,), _f32)
            return 0
        lax.fori_loop(0, RT, za16, 0)

        def count_chunk(t, _):
            pltpu.sync_copy(lst.at[w, pl.ds(t * K, K)], kbuf.at[pl.ds(0, K)])

            def one(k, _):
                dl = kbuf[pl.ds(k, 16)][0] & 511
                plsc.addupdate(acc16.at[dl, pl.ds(0, 16)],
                               jnp.ones((16,), _f32))
                return 0
            lax.fori_loop(0, K, one, 0)
            return 0
        lax.fori_loop(0, total // K, count_chunk, 0)
        pltpu.sync_copy(acc16.at[pl.ds(0, R)], deg.at[pl.ds(w * R, R)])


KB = 64


@functools.partial(
    pl.kernel,
    out_type=(
        jax.ShapeDtypeStruct((NR * D,), _f32),
        jax.ShapeDtypeStruct((NR * D,), _f32),
        jax.ShapeDtypeStruct((NR * D,), _f32),
    ),
    mesh=_mesh,
    scratch_types=[
        pltpu.VMEM((KB + 16,), _i32),
        pltpu.VMEM((KB + 16,), _i32),
        pltpu.VMEM((KB,), _i32),
        pltpu.VMEM((KB,), _i32),
        pltpu.VMEM((KB, D), _f32),
        pltpu.VMEM((KB, D), _f32),
        pltpu.VMEM((16,), _i32),
        pltpu.VMEM((RT * D,), _f32),
        pltpu.SemaphoreType.DMA,
        pltpu.SemaphoreType.DMA,
    ],
    compiler_params=pltpu.CompilerParams(needs_layout_passes=False),
)
def _sc_segsum3(h, hp, lst0, lst1, lst2, cnt0, cnt1, cnt2,
                o0, o1, o2, pkb0, pkb1, gsrc0, gsrc1, rows0, rows1,
                cntv, acc, sem0, sem1):
    c = lax.axis_index("c")
    s = lax.axis_index("s")
    w = c * 16 + s

    for tab, lst, cnt, out in ((hp, lst0, cnt0, o0),
                               (h, lst1, cnt1, o1),
                               (h, lst2, cnt2, o2)):
        def za(i, _):
            acc[pl.ds(i * 16, 16)] = jnp.zeros((16,), _f32)
            return 0
        lax.fori_loop(0, RT * 16, za, 0, unroll=4)

        pltpu.sync_copy(cnt.at[w], cntv)
        pairs = cntv[...][0] // (2 * KB)

        def mk_acc(pkb, rows):
            def one(k, _):
                dl = pkb[pl.ds(k, 16)][0] & 511
                b = dl << 8
                for cc in range(16):
                    plsc.addupdate(acc.at[pl.ds(b + cc * 16, 16)],
                                   rows[k, pl.ds(cc * 16, 16)])
                return 0
            return one

        def pair_chunk(t, _):
            e0 = t * 2 * KB
            pltpu.sync_copy(lst.at[w, pl.ds(e0, KB)], pkb0.at[pl.ds(0, KB)])

            def up0(j, _):
                gsrc0[pl.ds(j * 16, 16)] = pkb0[pl.ds(j * 16, 16)] >> 9
                return 0
            lax.fori_loop(0, KB // 16, up0, 0, unroll=4)
            d0 = pltpu.async_copy(tab.at[gsrc0], rows0, sem0)

            pltpu.sync_copy(lst.at[w, pl.ds(e0 + KB, KB)],
                            pkb1.at[pl.ds(0, KB)])

            def up1(j, _):
                gsrc1[pl.ds(j * 16, 16)] = pkb1[pl.ds(j * 16, 16)] >> 9
                return 0
            lax.fori_loop(0, KB // 16, up1, 0, unroll=4)
            d1 = pltpu.async_copy(tab.at[gsrc1], rows1, sem1)

            d0.wait()
            lax.fori_loop(0, KB, mk_acc(pkb0, rows0), 0, unroll=2)
            d1.wait()
            lax.fori_loop(0, KB, mk_acc(pkb1, rows1), 0, unroll=2)
            return 0
        lax.fori_loop(0, pairs, pair_chunk, 0)

        pltpu.sync_copy(acc.at[pl.ds(0, R * D)], out.at[pl.ds(w * R * D, R * D)])


def _prolog_body(x_ref, w_ref, b_ref, dga_ref, ca_ref, cb_ref,
                 h_ref, hp_ref, dinv_ref, ia_ref, ib_ref):
    h = jnp.maximum(jnp.dot(x_ref[...], w_ref[...],
                            preferred_element_type=_f32) + b_ref[...], 0.0)
    dinv = lax.rsqrt(dga_ref[...] + 1.0)
    h_ref[...] = h
    hp_ref[...] = h * dinv[:, None]
    dinv_ref[...] = dinv
    ia_ref[...] = 1.0 / jnp.maximum(ca_ref[...], 1.0)
    ib_ref[...] = 1.0 / jnp.maximum(cb_ref[...], 1.0)


def _tc_prolog(x, w, b, dga, ca, cb):
    blk = 256
    grid = pl.cdiv(N, blk)
    vspec = pl.BlockSpec((blk,), lambda i: (i,))
    return pl.pallas_call(
        _prolog_body,
        grid=(grid,),
        in_specs=[
            pl.BlockSpec((blk, D), lambda i: (i, 0)),
            pl.BlockSpec((D, D), lambda i: (0, 0)),
            pl.BlockSpec((D,), lambda i: (0,)),
            vspec, vspec, vspec,
        ],
        out_specs=[
            pl.BlockSpec((blk, D), lambda i: (i, 0)),
            pl.BlockSpec((blk, D), lambda i: (i, 0)),
            vspec, vspec, vspec,
        ],
        out_shape=[
            jax.ShapeDtypeStruct((N, D), _f32),
            jax.ShapeDtypeStruct((N, D), _f32),
            jax.ShapeDtypeStruct((N,), _f32),
            jax.ShapeDtypeStruct((N,), _f32),
            jax.ShapeDtypeStruct((N,), _f32),
        ],
    )(x, w, b, dga, ca, cb)


def _layer_body(ma_ref, md_ref, mc_ref, h_ref, hp_ref, dinv_ref, ia_ref,
                ib_ref, u_ref, bias_ref, g_ref, lb_ref, hn_ref, hpn_ref):
    dinv = dinv_ref[...][:, None]
    a = (ma_ref[...] + hp_ref[...]) * dinv
    b = md_ref[...] * ia_ref[...][:, None]
    cc = mc_ref[...] * ib_ref[...][:, None]
    x = jnp.concatenate([a, b, cc, h_ref[...]], axis=1)
    out = jnp.dot(x, u_ref[...], preferred_element_type=_f32) + bias_ref[...]
    out = jnp.maximum(out, 0.0)
    mu = jnp.mean(out, axis=1, keepdims=True)
    var = jnp.mean((out - mu) ** 2, axis=1, keepdims=True)
    hn = (out - mu) * lax.rsqrt(var + 1e-5) * g_ref[...] + lb_ref[...]
    hn_ref[...] = hn
    hpn_ref[...] = hn * dinv


def _tc_layer(ma, md, mc, h, hp, dinv, ia, ib, u, bias, g, lb):
    blk = 256
    grid = pl.cdiv(N, blk)
    mspec = pl.BlockSpec((blk, D), lambda i: (i, 0))
    vspec = pl.BlockSpec((blk,), lambda i: (i,))
    return pl.pallas_call(
        _layer_body,
        grid=(grid,),
        in_specs=[
            mspec, mspec, mspec, mspec, mspec,
            vspec, vspec, vspec,
            pl.BlockSpec((4 * D, D), lambda i: (0, 0)),
            pl.BlockSpec((D,), lambda i: (0,)),
            pl.BlockSpec((D,), lambda i: (0,)),
            pl.BlockSpec((D,), lambda i: (0,)),
        ],
        out_specs=[mspec, mspec],
        out_shape=[
            jax.ShapeDtypeStruct((N, D), _f32),
            jax.ShapeDtypeStruct((N, D), _f32),
        ],
    )(ma, md, mc, h, hp, dinv, ia, ib, u, bias, g, lb)


def _epi_body(h_ref, bn_ref, attw_ref, outw_ref, outb_ref, o_ref):
    h = h_ref[...]
    logits = jnp.dot(h, attw_ref[...], preferred_element_type=_f32)[:, 0]
    p = jnp.exp(logits - jnp.max(logits))
    p = p / jnp.sum(p)
    wtd = h * p[:, None]
    oh = (bn_ref[...][:, None] ==
          lax.broadcasted_iota(_i32, (N, G), 1)).astype(_f32)
    gr = lax.dot_general(oh, wtd, (((0,), (0,)), ((), ())),
                         preferred_element_type=_f32)
    gc = lax.dot_general(oh, p[:, None], (((0,), (0,)), ((), ())),
                         preferred_element_type=_f32)
    gr = gr / jnp.maximum(gc, 1e-6)
    o_ref[...] = jnp.dot(gr, outw_ref[...],
                         preferred_element_type=_f32) + outb_ref[...]


def _tc_epilogue(h, bn, attw, outw, outb):
    return pl.pallas_call(
        _epi_body,
        out_shape=jax.ShapeDtypeStruct((G, D), _f32),
    )(h, bn, attw, outw, outb)


def kernel(x_node, edge_index_ast, edge_index_df, edge_index_cf, batch_node,
           proj_W, proj_b, gcn_W, gcn_b, sgA_Wl, sgA_bl, sgA_Wr,
           sgB_Wl, sgB_bl, sgB_Wr, ln_g, ln_b, att_W, att_b, out_W, out_b):
    (lst0, lst1, lst2, cnt0, cnt1, cnt2,
     deg0, deg1, deg2) = _sc_bucketize(
         edge_index_ast[0], edge_index_ast[1],
         edge_index_df[0], edge_index_df[1],
         edge_index_cf[0], edge_index_cf[1])
    dga = deg0[:N, 0]
    ca = deg1[:N, 0]
    cb = deg2[:N, 0]
    import os as _os
    if _os.environ.get("_BISECT") == "A":
        return (dga[:G, None] + ca[:G, None] + cb[:G, None]
                + jnp.zeros((G, D), _f32)
                + lst0[0, 0] + cnt0[0, 0])

    h, hp, dinv, ia, ib = _tc_prolog(x_node, proj_W, proj_b, dga, ca, cb)

    for l in range(gcn_W.shape[0]):
        u = jnp.concatenate(
            [gcn_W[l], sgA_Wl[l], sgB_Wl[l], sgA_Wr[l] + sgB_Wr[l]], axis=0)
        bias = gcn_b[l] + sgA_bl[l] + sgB_bl[l]
        ma, md, mc = _sc_segsum3(h, hp, lst0, lst1, lst2, cnt0, cnt1, cnt2)
        ma = ma.reshape(NR, D)[:N]
        md = md.reshape(NR, D)[:N]
        mc = mc.reshape(NR, D)[:N]
        h, hp = _tc_layer(ma, md, mc, h, hp, dinv, ia, ib,
                          u, bias, ln_g, ln_b)

    return _tc_epilogue(h, batch_node, att_W, out_W, out_b)


# speedup vs baseline: 2.2988x; 1.1001x over previous
import functools

import jax
import jax.numpy as jnp
from jax import lax
from jax.experimental import pallas as pl
from jax.experimental.pallas import tpu as pltpu
from jax.experimental.pallas import tpu_sc as plsc

N = 10000
D = 256
G = 64
E = 160000
NB = 32
R = 320
NR = NB * R
RT = R + 8
K = 128
C = 2000
EPAD = E + K

_mesh = plsc.VectorSubcoreMesh(core_axis_name="c", subcore_axis_name="s",
                               num_cores=2, num_subcores=16)

_i32 = jnp.int32
_f32 = jnp.float32


@functools.partial(
    pl.kernel,
    out_type=(
        jax.ShapeDtypeStruct((NB, EPAD), _i32),
        jax.ShapeDtypeStruct((NB, EPAD), _i32),
        jax.ShapeDtypeStruct((NB, EPAD), _i32),
        jax.ShapeDtypeStruct((NB, 16), _i32),
        jax.ShapeDtypeStruct((NB, 16), _i32),
        jax.ShapeDtypeStruct((NB, 16), _i32),
        jax.ShapeDtypeStruct((NR, 16), _f32),
        jax.ShapeDtypeStruct((NR, 16), _f32),
        jax.ShapeDtypeStruct((NR, 16), _f32),
    ),
    mesh=_mesh,
    scratch_types=[
        pltpu.VMEM((C,), _i32),
        pltpu.VMEM((C,), _i32),
        pltpu.VMEM((C + 304,), _i32),
        pltpu.VMEM((16,), _i32),
        pltpu.VMEM((K + 16,), _i32),
        pltpu.VMEM((RT, 16), _f32),
    ],
    compiler_params=pltpu.CompilerParams(needs_layout_passes=False),
)
def _sc_bucketize(es0, ed0, es1, ed1, es2, ed2, lst0, lst1, lst2,
                  cnt0, cnt1, cnt2, deg0, deg1, deg2, srcb, dstb, listb,
                  cvec, kbuf, acc16):
    c = lax.axis_index("c")
    s = lax.axis_index("s")
    w = c * 16 + s
    lo = w * R

    for es, ed, lst, cnt, deg in ((es0, ed0, lst0, cnt0, deg0),
                                  (es1, ed1, lst1, cnt1, deg1),
                                  (es2, ed2, lst2, cnt2, deg2)):
        def chunk_body(t, carry):
            pos, outb = carry
            pltpu.sync_copy(es.at[pl.ds(t * C, C)], srcb)
            pltpu.sync_copy(ed.at[pl.ds(t * C, C)], dstb)

            def vec_body(j, posv):
                sv = srcb[pl.ds(j * 16, 16)]
                dv = dstb[pl.ds(j * 16, 16)]
                dl = dv - lo
                m = (dl >= 0) & (dl < R)
                pk = (sv << 9) | jnp.where(m, dl, 0)
                kin = m.astype(_i32)
                excl = plsc.cumsum(kin) - kin
                idx = jnp.where(m, posv + excl, C + 303)
                plsc.store_scatter(listb, [idx], pk)
                return posv + plsc.all_reduce_population_count(m)
            posv = lax.fori_loop(0, C // 16, vec_body,
                                 jnp.zeros((16,), _i32) + pos, unroll=4)
            pos = posv[0]

            nblk = pos // K

            def flush(b, _):
                pltpu.sync_copy(listb.at[pl.ds(b * K, K)],
                                lst.at[w, pl.ds((outb + b) * K, K)])
                return 0
            lax.fori_loop(0, nblk, flush, 0)
            rs = nblk * K

            def shift(v, _):
                tmp = listb[pl.ds(rs + v * 16, 16)]
                listb[pl.ds(v * 16, 16)] = tmp
                return 0
            lax.fori_loop(0, K // 16, shift, 0)
            return pos - rs, outb + nblk

        pos, outb = lax.fori_loop(0, E // C, chunk_body,
                                  (jnp.asarray(0, _i32), jnp.asarray(0, _i32)))

        sent = jnp.full((16,), R, _i32)

        def pad(v, _):
            listb[pl.ds(pos + v * 16, 16)] = sent
            return 0
        lax.fori_loop(0, K // 16, pad, 0)
        nblk2 = (pos + K - 1) // K

        def flush2(b, _):
            pltpu.sync_copy(listb.at[pl.ds(b * K, K)],
                            lst.at[w, pl.ds((outb + b) * K, K)])
            return 0
        lax.fori_loop(0, nblk2, flush2, 0)
        total = (outb + nblk2) * K
        cvec[...] = jnp.zeros((16,), _i32) + total
        pltpu.sync_copy(cvec, cnt.at[w])

        def za16(i, _):
            acc16[i, pl.ds(0, 16)] = jnp.zeros((16,), _f32)
            return 0
        lax.fori_loop(0, RT, za16, 0)

        def count_chunk(t, _):
            pltpu.sync_copy(lst.at[w, pl.ds(t * K, K)], kbuf.at[pl.ds(0, K)])

            def one(k, _):
                dl = kbuf[pl.ds(k, 16)][0] & 511
                plsc.addupdate(acc16.at[dl, pl.ds(0, 16)],
                               jnp.ones((16,), _f32))
                return 0
            lax.fori_loop(0, K, one, 0)
            return 0
        lax.fori_loop(0, total // K, count_chunk, 0)
        pltpu.sync_copy(acc16.at[pl.ds(0, R)], deg.at[pl.ds(w * R, R)])


KB = 64


@functools.partial(
    pl.kernel,
    out_type=(
        jax.ShapeDtypeStruct((NR * D,), _f32),
        jax.ShapeDtypeStruct((NR * D,), _f32),
        jax.ShapeDtypeStruct((NR * D,), _f32),
    ),
    mesh=_mesh,
    scratch_types=[
        pltpu.VMEM((KB + 16,), _i32),
        pltpu.VMEM((KB + 16,), _i32),
        pltpu.VMEM((KB,), _i32),
        pltpu.VMEM((KB,), _i32),
        pltpu.VMEM((KB, D), _f32),
        pltpu.VMEM((KB, D), _f32),
        pltpu.VMEM((16,), _i32),
        pltpu.VMEM((RT * D,), _f32),
        pltpu.SMEM((KB,), _i32),
        pltpu.SMEM((KB,), _i32),
        pltpu.SemaphoreType.DMA,
        pltpu.SemaphoreType.DMA,
    ],
    compiler_params=pltpu.CompilerParams(needs_layout_passes=False),
)
def _sc_segsum3(h, hp, lst0, lst1, lst2, cnt0, cnt1, cnt2,
                o0, o1, o2, pkb0, pkb1, gsrc0, gsrc1, rows0, rows1,
                cntv, acc, sm0, sm1, sem0, sem1):
    c = lax.axis_index("c")
    s = lax.axis_index("s")
    w = c * 16 + s

    for tab, lst, cnt, out in ((hp, lst0, cnt0, o0),
                               (h, lst1, cnt1, o1),
                               (h, lst2, cnt2, o2)):
        def za(i, _):
            acc[pl.ds(i * 16, 16)] = jnp.zeros((16,), _f32)
            return 0
        lax.fori_loop(0, RT * 16, za, 0, unroll=4)

        pltpu.sync_copy(cnt.at[w], cntv)
        pairs = cntv[...][0] // (2 * KB)

        def mk_acc(sm, rows):
            def one(k, _):
                b = (sm[k] & 511) << 8
                for cc in range(16):
                    plsc.addupdate(acc.at[pl.ds(b + cc * 16, 16)],
                                   rows[k, pl.ds(cc * 16, 16)])
                return 0
            return one

        def mk_ext(pkb, sm):
            def ext(k, _):
                sm[k] = pkb[pl.ds(k, 16)][0]
                return 0
            return ext

        def pair_chunk(t, _):
            e0 = t * 2 * KB
            pltpu.sync_copy(lst.at[w, pl.ds(e0, KB)], pkb0.at[pl.ds(0, KB)])

            def up0(j, _):
                gsrc0[pl.ds(j * 16, 16)] = pkb0[pl.ds(j * 16, 16)] >> 9
                return 0
            lax.fori_loop(0, KB // 16, up0, 0, unroll=4)
            d0 = pltpu.async_copy(tab.at[gsrc0], rows0, sem0)
            lax.fori_loop(0, KB, mk_ext(pkb0, sm0), 0, unroll=8)

            pltpu.sync_copy(lst.at[w, pl.ds(e0 + KB, KB)],
                            pkb1.at[pl.ds(0, KB)])

            def up1(j, _):
                gsrc1[pl.ds(j * 16, 16)] = pkb1[pl.ds(j * 16, 16)] >> 9
                return 0
            lax.fori_loop(0, KB // 16, up1, 0, unroll=4)
            d1 = pltpu.async_copy(tab.at[gsrc1], rows1, sem1)
            lax.fori_loop(0, KB, mk_ext(pkb1, sm1), 0, unroll=8)

            d0.wait()
            lax.fori_loop(0, KB, mk_acc(sm0, rows0), 0, unroll=2)
            d1.wait()
            lax.fori_loop(0, KB, mk_acc(sm1, rows1), 0, unroll=2)
            return 0
        lax.fori_loop(0, pairs, pair_chunk, 0)

        pltpu.sync_copy(acc.at[pl.ds(0, R * D)], out.at[pl.ds(w * R * D, R * D)])


def _prolog_body(x_ref, w_ref, b_ref, dga_ref, ca_ref, cb_ref,
                 h_ref, hp_ref, dinv_ref, ia_ref, ib_ref):
    h = jnp.maximum(jnp.dot(x_ref[...], w_ref[...],
                            preferred_element_type=_f32) + b_ref[...], 0.0)
    dinv = lax.rsqrt(dga_ref[...] + 1.0)
    h_ref[...] = h
    hp_ref[...] = h * dinv[:, None]
    dinv_ref[...] = dinv
    ia_ref[...] = 1.0 / jnp.maximum(ca_ref[...], 1.0)
    ib_ref[...] = 1.0 / jnp.maximum(cb_ref[...], 1.0)


def _tc_prolog(x, w, b, dga, ca, cb):
    blk = 256
    grid = pl.cdiv(N, blk)
    vspec = pl.BlockSpec((blk,), lambda i: (i,))
    return pl.pallas_call(
        _prolog_body,
        grid=(grid,),
        in_specs=[
            pl.BlockSpec((blk, D), lambda i: (i, 0)),
            pl.BlockSpec((D, D), lambda i: (0, 0)),
            pl.BlockSpec((D,), lambda i: (0,)),
            vspec, vspec, vspec,
        ],
        out_specs=[
            pl.BlockSpec((blk, D), lambda i: (i, 0)),
            pl.BlockSpec((blk, D), lambda i: (i, 0)),
            vspec, vspec, vspec,
        ],
        out_shape=[
            jax.ShapeDtypeStruct((N, D), _f32),
            jax.ShapeDtypeStruct((N, D), _f32),
            jax.ShapeDtypeStruct((N,), _f32),
            jax.ShapeDtypeStruct((N,), _f32),
            jax.ShapeDtypeStruct((N,), _f32),
        ],
    )(x, w, b, dga, ca, cb)


def _layer_body(ma_ref, md_ref, mc_ref, h_ref, hp_ref, dinv_ref, ia_ref,
                ib_ref, u_ref, bias_ref, g_ref, lb_ref, hn_ref, hpn_ref):
    dinv = dinv_ref[...][:, None]
    a = (ma_ref[...] + hp_ref[...]) * dinv
    b = md_ref[...] * ia_ref[...][:, None]
    cc = mc_ref[...] * ib_ref[...][:, None]
    x = jnp.concatenate([a, b, cc, h_ref[...]], axis=1)
    out = jnp.dot(x, u_ref[...], preferred_element_type=_f32) + bias_ref[...]
    out = jnp.maximum(out, 0.0)
    mu = jnp.mean(out, axis=1, keepdims=True)
    var = jnp.mean((out - mu) ** 2, axis=1, keepdims=True)
    hn = (out - mu) * lax.rsqrt(var + 1e-5) * g_ref[...] + lb_ref[...]
    hn_ref[...] = hn
    hpn_ref[...] = hn * dinv


def _tc_layer(ma, md, mc, h, hp, dinv, ia, ib, u, bias, g, lb):
    blk = 256
    grid = pl.cdiv(N, blk)
    mspec = pl.BlockSpec((blk, D), lambda i: (i, 0))
    vspec = pl.BlockSpec((blk,), lambda i: (i,))
    return pl.pallas_call(
        _layer_body,
        grid=(grid,),
        in_specs=[
            mspec, mspec, mspec, mspec, mspec,
            vspec, vspec, vspec,
            pl.BlockSpec((4 * D, D), lambda i: (0, 0)),
            pl.BlockSpec((D,), lambda i: (0,)),
            pl.BlockSpec((D,), lambda i: (0,)),
            pl.BlockSpec((D,), lambda i: (0,)),
        ],
        out_specs=[mspec, mspec],
        out_shape=[
            jax.ShapeDtypeStruct((N, D), _f32),
            jax.ShapeDtypeStruct((N, D), _f32),
        ],
    )(ma, md, mc, h, hp, dinv, ia, ib, u, bias, g, lb)


def _epi_body(h_ref, bn_ref, attw_ref, outw_ref, outb_ref, o_ref):
    h = h_ref[...]
    logits = jnp.dot(h, attw_ref[...], preferred_element_type=_f32)[:, 0]
    p = jnp.exp(logits - jnp.max(logits))
    p = p / jnp.sum(p)
    wtd = h * p[:, None]
    oh = (bn_ref[...][:, None] ==
          lax.broadcasted_iota(_i32, (N, G), 1)).astype(_f32)
    gr = lax.dot_general(oh, wtd, (((0,), (0,)), ((), ())),
                         preferred_element_type=_f32)
    gc = lax.dot_general(oh, p[:, None], (((0,), (0,)), ((), ())),
                         preferred_element_type=_f32)
    gr = gr / jnp.maximum(gc, 1e-6)
    o_ref[...] = jnp.dot(gr, outw_ref[...],
                         preferred_element_type=_f32) + outb_ref[...]


def _tc_epilogue(h, bn, attw, outw, outb):
    return pl.pallas_call(
        _epi_body,
        out_shape=jax.ShapeDtypeStruct((G, D), _f32),
    )(h, bn, attw, outw, outb)


def kernel(x_node, edge_index_ast, edge_index_df, edge_index_cf, batch_node,
           proj_W, proj_b, gcn_W, gcn_b, sgA_Wl, sgA_bl, sgA_Wr,
           sgB_Wl, sgB_bl, sgB_Wr, ln_g, ln_b, att_W, att_b, out_W, out_b):
    (lst0, lst1, lst2, cnt0, cnt1, cnt2,
     deg0, deg1, deg2) = _sc_bucketize(
         edge_index_ast[0], edge_index_ast[1],
         edge_index_df[0], edge_index_df[1],
         edge_index_cf[0], edge_index_cf[1])
    dga = deg0[:N, 0]
    ca = deg1[:N, 0]
    cb = deg2[:N, 0]
    import os as _os
    if _os.environ.get("_BISECT") == "A":
        return (dga[:G, None] + ca[:G, None] + cb[:G, None]
                + jnp.zeros((G, D), _f32)
                + lst0[0, 0] + cnt0[0, 0])

    h, hp, dinv, ia, ib = _tc_prolog(x_node, proj_W, proj_b, dga, ca, cb)

    for l in range(gcn_W.shape[0]):
        u = jnp.concatenate(
            [gcn_W[l], sgA_Wl[l], sgB_Wl[l], sgA_Wr[l] + sgB_Wr[l]], axis=0)
        bias = gcn_b[l] + sgA_bl[l] + sgB_bl[l]
        ma, md, mc = _sc_segsum3(h, hp, lst0, lst1, lst2, cnt0, cnt1, cnt2)
        ma = ma.reshape(NR, D)[:N]
        md = md.reshape(NR, D)[:N]
        mc = mc.reshape(NR, D)[:N]
        h, hp = _tc_layer(ma, md, mc, h, hp, dinv, ia, ib,
                          u, bias, ln_g, ln_b)

    return _tc_epilogue(h, batch_node, att_W, out_W, out_b)
